# Initial kernel scaffold; baseline (speedup 1.0000x reference)
import jax
import jax.numpy as jnp
from jax.experimental import pallas as pl


def kernel(h, big_features, edge_index, edge_weight, g_mean, be_mean, g_std, be_std, g_edge, be_edge, g_n1, be_n1, W_sum, b_sum, W_meanagg, b_meanagg, W_num, b_num, W_fc1, b_fc1, W_fc2, b_fc2):
    raise NotImplementedError("write your pallas kernel here")


# speedup vs baseline: 8.0882x; 8.0882x over previous
import functools

import jax
import jax.numpy as jnp
from jax import lax
from jax.experimental import pallas as pl
from jax.experimental.pallas import tpu as pltpu
from jax.experimental.pallas import tpu_sc as plsc

_N = 100000
_E = 3200000
_F = 20

_NT = 16
_C = 256
_NQ = _E // _C

_NP = 100096
_COLS_PER_TILE = 6400
_COLS_LAST = _NP - 15 * _COLS_PER_TILE
_CS = 256

_CCHUNK = 2000
_EPT_CNT = _E // 2 // _NT
_NCCHUNK = _EPT_CNT // _CCHUNK

_BLK = 2000
_NBLK = _N // _BLK


def _pre_body(h_ref, bf_ref, g_ref, be_ref, o_ref):
    bf = bf_ref[...]
    g = g_ref[...]
    be = be_ref[...]
    parts = [h_ref[...]]
    for c in range(3):
        blk = bf[:, 5 * c:5 * c + 5]
        mu = jnp.mean(blk, axis=-1, keepdims=True)
        var = jnp.mean((blk - mu) ** 2, axis=-1, keepdims=True)
        nrm = (blk - mu) / jnp.sqrt(var + 1e-5)
        parts.append(nrm * g[:, 5 * c:5 * c + 5] + be[:, 5 * c:5 * c + 5])
    o_ref[...] = jnp.concatenate(parts, axis=-1)


def _pre_call(h, bf, g, be):
    return pl.pallas_call(
        _pre_body,
        grid=(_NBLK,),
        in_specs=[
            pl.BlockSpec((_BLK, 5), lambda i: (i, 0)),
            pl.BlockSpec((_BLK, 15), lambda i: (i, 0)),
            pl.BlockSpec((1, 15), lambda i: (0, 0)),
            pl.BlockSpec((1, 15), lambda i: (0, 0)),
        ],
        out_specs=pl.BlockSpec((_BLK, _F), lambda i: (i, 0)),
        out_shape=jax.ShapeDtypeStruct((_N, _F), jnp.float32),
    )(h, bf, g, be)


def _g_body(xt_hbm, src_hbm, xg_hbm, x_sh, src_v, blk_v, stg_v):
    cid = lax.axis_index("c")
    sid = lax.axis_index("s")
    t = cid * _NT + sid
    col0 = sid * _COLS_PER_TILE
    nstage = lax.select(sid < 15, _COLS_PER_TILE // _CS, _COLS_LAST // _CS)

    for f in range(_F):
        def sput(i, c, f=f):
            o = col0 + i * _CS
            pltpu.sync_copy(xt_hbm.at[f].at[pl.ds(o, _CS)], stg_v)
            pltpu.sync_copy(stg_v, x_sh.at[f].at[pl.ds(o, _CS)])
            return c
        lax.fori_loop(0, nstage, sput, 0)

    plsc.subcore_barrier()

    nq = (_NQ - t + 31) // 32

    def chunk(k, c):
        q = t + k * 32
        pltpu.sync_copy(src_hbm.at[pl.ds(q * _C, _C)], src_v)
        for f in range(_F):
            pltpu.sync_copy(x_sh.at[f].at[src_v], blk_v.at[f])
        pltpu.sync_copy(blk_v, xg_hbm.at[q])
        return c
    lax.fori_loop(0, nq, chunk, 0)


def _a_body(xg_hbm, dst_hbm, ew_hbm, awt_hbm, aut_hbm,
            acc_sh, blk_v, dst_v, ew_v, stg_v):
    cid = lax.axis_index("c")
    sid = lax.axis_index("s")
    col0 = sid * _COLS_PER_TILE
    nstage = lax.select(sid < 15, _COLS_PER_TILE // _CS, _COLS_LAST // _CS)

    def zput(i, c):
        stg_v[pl.ds(i * 16, 16)] = jnp.zeros((16,), jnp.float32)
        return c
    lax.fori_loop(0, _CS // 16, zput, 0)

    for f in range(_F):
        def zcol(i, c, f=f):
            pltpu.sync_copy(stg_v,
                            acc_sh.at[f].at[pl.ds(col0 + i * _CS, _CS)])
            return c
        lax.fori_loop(0, nstage, zcol, 0)

    plsc.subcore_barrier()

    nq = (_NQ - sid + 15) // 16

    def chunk(k, c):
        q = sid + k * 16
        b = q * _C
        pltpu.sync_copy(xg_hbm.at[q], blk_v)
        pltpu.sync_copy(dst_hbm.at[pl.ds(b, _C)], dst_v)

        @pl.when(cid == 0)
        def _():
            pltpu.sync_copy(ew_hbm.at[pl.ds(b, _C)], ew_v)

            def mul(g, c2):
                wv = ew_v[pl.ds(g * 16, 16)]
                for f in range(_F):
                    blk_v[f, pl.ds(g * 16, 16)] = (
                        blk_v[f, pl.ds(g * 16, 16)] * wv)
                return c2
            lax.fori_loop(0, _C // 16, mul, 0)

        for f in range(_F):
            pltpu.sync_copy(blk_v.at[f], acc_sh.at[f].at[dst_v], add=True)
        return c
    lax.fori_loop(0, nq, chunk, 0)

    plsc.subcore_barrier()

    def out_to(out_hbm):
        for f in range(_F):
            def ocol(i, c, f=f):
                o = col0 + i * _CS
                pltpu.sync_copy(acc_sh.at[f].at[pl.ds(o, _CS)], stg_v)
                pltpu.sync_copy(stg_v, out_hbm.at[f].at[pl.ds(o, _CS)])
                return c
            lax.fori_loop(0, nstage, ocol, 0)

    @pl.when(cid == 0)
    def _():
        out_to(awt_hbm)

    @pl.when(cid == 1)
    def _():
        out_to(aut_hbm)


def _cnt_body(dst_hbm, c0_hbm, c1_hbm, cnt_sh, dst_v, one_v, stg_v):
    cid = lax.axis_index("c")
    sid = lax.axis_index("s")
    row0 = sid * _COLS_PER_TILE
    nstage = lax.select(sid < 15, _COLS_PER_TILE // _CS, _COLS_LAST // _CS)

    def zput(i, c):
        stg_v[pl.ds(i * 16, 16)] = jnp.zeros((16,), jnp.float32)
        return c
    lax.fori_loop(0, _CS // 16, zput, 0)

    def oput(i, c):
        one_v[pl.ds(i * 16, 16)] = jnp.ones((16,), jnp.float32)
        return c
    lax.fori_loop(0, _CCHUNK // 16, oput, 0)

    def zchunk(i, c):
        pltpu.sync_copy(stg_v, cnt_sh.at[pl.ds(row0 + i * _CS, _CS)])
        return c
    lax.fori_loop(0, nstage, zchunk, 0)

    plsc.subcore_barrier()

    ebase = (cid * _NT + sid) * _EPT_CNT

    def chunk(k, c):
        b = ebase + k * _CCHUNK
        pltpu.sync_copy(dst_hbm.at[pl.ds(b, _CCHUNK)], dst_v)
        pltpu.sync_copy(one_v, cnt_sh.at[dst_v], add=True)
        return c
    lax.fori_loop(0, _NCCHUNK, chunk, 0)

    plsc.subcore_barrier()

    def out_to(out_hbm):
        def go(i, c):
            r = row0 + i * _CS
            pltpu.sync_copy(cnt_sh.at[pl.ds(r, _CS)], stg_v)
            pltpu.sync_copy(stg_v, out_hbm.at[pl.ds(r, _CS)])
            return c
        return go

    @pl.when(cid == 0)
    def _():
        lax.fori_loop(0, nstage, out_to(c0_hbm), 0)

    @pl.when(cid == 1)
    def _():
        lax.fori_loop(0, nstage, out_to(c1_hbm), 0)


_sc_cache = {}
_MESH = dict(core_axis_name="c", subcore_axis_name="s",
             num_cores=2, num_subcores=_NT)


def _sc_g(*args):
    if "g" not in _sc_cache:
        _sc_cache["g"] = functools.partial(
            pl.kernel,
            out_type=jax.ShapeDtypeStruct((_NQ, _F, _C), jnp.float32),
            mesh=plsc.VectorSubcoreMesh(**_MESH),
            scratch_types=[
                pltpu.VMEM_SHARED((_F, _NP), jnp.float32),
                pltpu.VMEM((_C,), jnp.int32),
                pltpu.VMEM((_F, _C), jnp.float32),
                pltpu.VMEM((_CS,), jnp.float32),
            ],
            compiler_params=pltpu.CompilerParams(use_tc_tiling_on_sc=False),
        )(_g_body)
    return _sc_cache["g"](*args)


def _sc_a(*args):
    if "a" not in _sc_cache:
        _sc_cache["a"] = functools.partial(
            pl.kernel,
            out_type=(
                jax.ShapeDtypeStruct((_F, _NP), jnp.float32),
                jax.ShapeDtypeStruct((_F, _NP), jnp.float32),
            ),
            mesh=plsc.VectorSubcoreMesh(**_MESH),
            scratch_types=[
                pltpu.VMEM_SHARED((_F, _NP), jnp.float32),
                pltpu.VMEM((_F, _C), jnp.float32),
                pltpu.VMEM((_C,), jnp.int32),
                pltpu.VMEM((_C,), jnp.float32),
                pltpu.VMEM((_CS,), jnp.float32),
            ],
            compiler_params=pltpu.CompilerParams(use_tc_tiling_on_sc=False),
        )(_a_body)
    return _sc_cache["a"](*args)


def _sc_cnt(*args):
    if "cnt" not in _sc_cache:
        _sc_cache["cnt"] = functools.partial(
            pl.kernel,
            out_type=(
                jax.ShapeDtypeStruct((_NP,), jnp.float32),
                jax.ShapeDtypeStruct((_NP,), jnp.float32),
            ),
            mesh=plsc.VectorSubcoreMesh(**_MESH),
            scratch_types=[
                pltpu.VMEM_SHARED((_NP,), jnp.float32),
                pltpu.VMEM((_CCHUNK,), jnp.int32),
                pltpu.VMEM((_CCHUNK,), jnp.float32),
                pltpu.VMEM((_CS,), jnp.float32),
            ],
            compiler_params=pltpu.CompilerParams(use_tc_tiling_on_sc=False),
        )(_cnt_body)
    return _sc_cache["cnt"](*args)


def _post_body(aw_ref, au_ref, c0_ref, c1_ref, ws_ref, bs_ref, wm_ref,
               bm_ref, wn_ref, bn_ref, g_ref, be_ref, w1_ref, b1_ref,
               w2_ref, b2_ref, o_ref):
    aw = aw_ref[...]
    au = au_ref[...]
    cnt = jnp.maximum(c0_ref[...] + c1_ref[...], 1.0)
    dot = functools.partial(lax.dot_general,
                            dimension_numbers=(((1,), (0,)), ((), ())),
                            preferred_element_type=jnp.float32)
    h1 = jax.nn.relu(dot(aw, ws_ref[...]) + bs_ref[...])
    h2 = jax.nn.relu(dot(aw, wm_ref[...]) / cnt + bm_ref[...])
    h3 = jax.nn.relu(dot(au, wn_ref[...]) + bn_ref[...])
    hh = jnp.concatenate([h1, h2, h3], axis=-1)
    mu = jnp.mean(hh, axis=-1, keepdims=True)
    var = jnp.mean((hh - mu) ** 2, axis=-1, keepdims=True)
    hh = (hh - mu) / jnp.sqrt(var + 1e-5) * g_ref[...] + be_ref[...]
    hh = jax.nn.relu(dot(hh, w1_ref[...]) + b1_ref[...])
    o_ref[...] = dot(hh, w2_ref[...]) + b2_ref[...]


def _post_call(aw, au, c0, c1, ws, bs, wm, bm, wn, bn, g, be, w1, b1, w2, b2):
    full = lambda r, c: pl.BlockSpec((r, c), lambda i: (0, 0))
    return pl.pallas_call(
        _post_body,
        grid=(_NBLK,),
        in_specs=[
            pl.BlockSpec((_BLK, _F), lambda i: (i, 0)),
            pl.BlockSpec((_BLK, _F), lambda i: (i, 0)),
            pl.BlockSpec((_BLK, 1), lambda i: (i, 0)),
            pl.BlockSpec((_BLK, 1), lambda i: (i, 0)),
            full(20, 64), full(1, 64), full(20, 64), full(1, 64),
            full(20, 64), full(1, 64), full(1, 192), full(1, 192),
            full(192, 192), full(1, 192), full(192, 5), full(1, 5),
        ],
        out_specs=pl.BlockSpec((_BLK, 5), lambda i: (i, 0)),
        out_shape=jax.ShapeDtypeStruct((_N, 5), jnp.float32),
    )(aw, au, c0, c1, ws, bs, wm, bm, wn, bn, g, be, w1, b1, w2, b2)


def kernel(h, big_features, edge_index, edge_weight,
           g_mean, be_mean, g_std, be_std, g_edge, be_edge, g_n1, be_n1,
           W_sum, b_sum, W_meanagg, b_meanagg, W_num, b_num,
           W_fc1, b_fc1, W_fc2, b_fc2):
    g = jnp.concatenate([g_mean, g_std, g_edge]).reshape(1, 15)
    be = jnp.concatenate([be_mean, be_std, be_edge]).reshape(1, 15)
    x = _pre_call(h, big_features, g, be)
    xt = jnp.pad(x.T, ((0, 0), (0, _NP - _N)))

    src = edge_index[0]
    dst = edge_index[1]
    xg = _sc_g(xt, src)
    awt, aut = _sc_a(xg, dst, edge_weight)

    dst_dep, awt = lax.optimization_barrier((dst, awt))
    c0, c1 = _sc_cnt(dst_dep)

    aw = awt[:, :_N].T
    au = aut[:, :_N].T
    c0 = c0[:_N].reshape(_N, 1)
    c1 = c1[:_N].reshape(_N, 1)
    return _post_call(
        aw, au, c0, c1,
        W_sum, b_sum.reshape(1, 64), W_meanagg, b_meanagg.reshape(1, 64),
        W_num, b_num.reshape(1, 64), g_n1.reshape(1, 192),
        be_n1.reshape(1, 192), W_fc1, b_fc1.reshape(1, 192),
        W_fc2, b_fc2.reshape(1, 5))


# speedup vs baseline: 8.2744x; 1.0230x over previous
import functools

import jax
import jax.numpy as jnp
from jax import lax
from jax.experimental import pallas as pl
from jax.experimental.pallas import tpu as pltpu
from jax.experimental.pallas import tpu_sc as plsc

_N = 100000
_E = 3200000
_F = 20

_NT = 16
_C = 128
_NQ = _E // _C

_NP = 100096
_COLS_PER_TILE = 6400
_COLS_LAST = _NP - 15 * _COLS_PER_TILE
_CS = 256

_CCHUNK = 2000
_EPT_CNT = _E // 2 // _NT
_NCCHUNK = _EPT_CNT // _CCHUNK

_BLK = 2000
_NBLK = _N // _BLK


def _pre_body(h_ref, bf_ref, g_ref, be_ref, o_ref):
    bf = bf_ref[...]
    g = g_ref[...]
    be = be_ref[...]
    parts = [h_ref[...]]
    for c in range(3):
        blk = bf[:, 5 * c:5 * c + 5]
        mu = jnp.mean(blk, axis=-1, keepdims=True)
        var = jnp.mean((blk - mu) ** 2, axis=-1, keepdims=True)
        nrm = (blk - mu) / jnp.sqrt(var + 1e-5)
        parts.append(nrm * g[:, 5 * c:5 * c + 5] + be[:, 5 * c:5 * c + 5])
    o_ref[...] = jnp.concatenate(parts, axis=-1)


def _pre_call(h, bf, g, be):
    return pl.pallas_call(
        _pre_body,
        grid=(_NBLK,),
        in_specs=[
            pl.BlockSpec((_BLK, 5), lambda i: (i, 0)),
            pl.BlockSpec((_BLK, 15), lambda i: (i, 0)),
            pl.BlockSpec((1, 15), lambda i: (0, 0)),
            pl.BlockSpec((1, 15), lambda i: (0, 0)),
        ],
        out_specs=pl.BlockSpec((_BLK, _F), lambda i: (i, 0)),
        out_shape=jax.ShapeDtypeStruct((_N, _F), jnp.float32),
    )(h, bf, g, be)


def _g_body(xt_hbm, src_hbm, xg_hbm, x_sh, src_v, idx_v, blk_v, stg_v):
    cid = lax.axis_index("c")
    sid = lax.axis_index("s")
    t = cid * _NT + sid
    col0 = sid * _COLS_PER_TILE
    nstage = lax.select(sid < 15, _COLS_PER_TILE // _CS, _COLS_LAST // _CS)

    for f in range(_F):
        def sput(i, c, f=f):
            o = col0 + i * _CS
            pltpu.sync_copy(xt_hbm.at[f].at[pl.ds(o, _CS)], stg_v)
            pltpu.sync_copy(stg_v, x_sh.at[pl.ds(f * _NP + o, _CS)])
            return c
        lax.fori_loop(0, nstage, sput, 0)

    plsc.subcore_barrier()

    nq = (_NQ - t + 31) // 32

    def chunk(k, c):
        q = t + k * 32
        pltpu.sync_copy(src_hbm.at[pl.ds(q * _C, _C)], src_v)

        def mkidx(g, c2):
            d = src_v[pl.ds(g * 16, 16)]
            for f in range(_F):
                idx_v[pl.ds(f * _C + g * 16, 16)] = d + f * _NP
            return c2
        lax.fori_loop(0, _C // 16, mkidx, 0)
        pltpu.sync_copy(x_sh.at[idx_v], blk_v)
        pltpu.sync_copy(blk_v, xg_hbm.at[q])
        return c
    lax.fori_loop(0, nq, chunk, 0)


def _a_body(xg_hbm, dst_hbm, ew_hbm, awt_hbm, aut_hbm,
            acc_sh, blk_v, dst_v, idx_v, ew_v, stg_v):
    cid = lax.axis_index("c")
    sid = lax.axis_index("s")
    col0 = sid * _COLS_PER_TILE
    nstage = lax.select(sid < 15, _COLS_PER_TILE // _CS, _COLS_LAST // _CS)

    def zput(i, c):
        stg_v[pl.ds(i * 16, 16)] = jnp.zeros((16,), jnp.float32)
        return c
    lax.fori_loop(0, _CS // 16, zput, 0)

    for f in range(_F):
        def zcol(i, c, f=f):
            pltpu.sync_copy(
                stg_v, acc_sh.at[pl.ds(f * _NP + col0 + i * _CS, _CS)])
            return c
        lax.fori_loop(0, nstage, zcol, 0)

    plsc.subcore_barrier()

    nq = (_NQ - sid + 15) // 16

    def chunk(k, c):
        q = sid + k * 16
        b = q * _C
        pltpu.sync_copy(xg_hbm.at[q], blk_v)
        pltpu.sync_copy(dst_hbm.at[pl.ds(b, _C)], dst_v)

        def mkidx(g, c2):
            d = dst_v[pl.ds(g * 16, 16)]
            for f in range(_F):
                idx_v[pl.ds(f * _C + g * 16, 16)] = d + f * _NP
            return c2
        lax.fori_loop(0, _C // 16, mkidx, 0)

        @pl.when(cid == 0)
        def _():
            pltpu.sync_copy(ew_hbm.at[pl.ds(b, _C)], ew_v)

            def mul(g, c2):
                wv = ew_v[pl.ds(g * 16, 16)]
                for f in range(_F):
                    blk_v[pl.ds(f * _C + g * 16, 16)] = (
                        blk_v[pl.ds(f * _C + g * 16, 16)] * wv)
                return c2
            lax.fori_loop(0, _C // 16, mul, 0)

        pltpu.sync_copy(blk_v, acc_sh.at[idx_v], add=True)
        return c
    lax.fori_loop(0, nq, chunk, 0)

    plsc.subcore_barrier()

    def out_to(out_hbm):
        for f in range(_F):
            def ocol(i, c, f=f):
                o = col0 + i * _CS
                pltpu.sync_copy(acc_sh.at[pl.ds(f * _NP + o, _CS)], stg_v)
                pltpu.sync_copy(stg_v, out_hbm.at[pl.ds(f * _NP + o, _CS)])
                return c
            lax.fori_loop(0, nstage, ocol, 0)

    @pl.when(cid == 0)
    def _():
        out_to(awt_hbm)

    @pl.when(cid == 1)
    def _():
        out_to(aut_hbm)


def _cnt_body(dst_hbm, c0_hbm, c1_hbm, cnt_sh, dst_v, one_v, stg_v):
    cid = lax.axis_index("c")
    sid = lax.axis_index("s")
    row0 = sid * _COLS_PER_TILE
    nstage = lax.select(sid < 15, _COLS_PER_TILE // _CS, _COLS_LAST // _CS)

    def zput(i, c):
        stg_v[pl.ds(i * 16, 16)] = jnp.zeros((16,), jnp.float32)
        return c
    lax.fori_loop(0, _CS // 16, zput, 0)

    def oput(i, c):
        one_v[pl.ds(i * 16, 16)] = jnp.ones((16,), jnp.float32)
        return c
    lax.fori_loop(0, _CCHUNK // 16, oput, 0)

    def zchunk(i, c):
        pltpu.sync_copy(stg_v, cnt_sh.at[pl.ds(row0 + i * _CS, _CS)])
        return c
    lax.fori_loop(0, nstage, zchunk, 0)

    plsc.subcore_barrier()

    ebase = (cid * _NT + sid) * _EPT_CNT

    def chunk(k, c):
        b = ebase + k * _CCHUNK
        pltpu.sync_copy(dst_hbm.at[pl.ds(b, _CCHUNK)], dst_v)
        pltpu.sync_copy(one_v, cnt_sh.at[dst_v], add=True)
        return c
    lax.fori_loop(0, _NCCHUNK, chunk, 0)

    plsc.subcore_barrier()

    def out_to(out_hbm):
        def go(i, c):
            r = row0 + i * _CS
            pltpu.sync_copy(cnt_sh.at[pl.ds(r, _CS)], stg_v)
            pltpu.sync_copy(stg_v, out_hbm.at[pl.ds(r, _CS)])
            return c
        return go

    @pl.when(cid == 0)
    def _():
        lax.fori_loop(0, nstage, out_to(c0_hbm), 0)

    @pl.when(cid == 1)
    def _():
        lax.fori_loop(0, nstage, out_to(c1_hbm), 0)


_sc_cache = {}
_MESH = dict(core_axis_name="c", subcore_axis_name="s",
             num_cores=2, num_subcores=_NT)


def _sc_g(*args):
    if "g" not in _sc_cache:
        _sc_cache["g"] = functools.partial(
            pl.kernel,
            out_type=jax.ShapeDtypeStruct((_NQ, _F * _C), jnp.float32),
            mesh=plsc.VectorSubcoreMesh(**_MESH),
            scratch_types=[
                pltpu.VMEM_SHARED((_F * _NP,), jnp.float32),
                pltpu.VMEM((_C,), jnp.int32),
                pltpu.VMEM((_F * _C,), jnp.int32),
                pltpu.VMEM((_F * _C,), jnp.float32),
                pltpu.VMEM((_CS,), jnp.float32),
            ],
            compiler_params=pltpu.CompilerParams(use_tc_tiling_on_sc=False),
        )(_g_body)
    return _sc_cache["g"](*args)


def _sc_a(*args):
    if "a" not in _sc_cache:
        _sc_cache["a"] = functools.partial(
            pl.kernel,
            out_type=(
                jax.ShapeDtypeStruct((_F * _NP,), jnp.float32),
                jax.ShapeDtypeStruct((_F * _NP,), jnp.float32),
            ),
            mesh=plsc.VectorSubcoreMesh(**_MESH),
            scratch_types=[
                pltpu.VMEM_SHARED((_F * _NP,), jnp.float32),
                pltpu.VMEM((_F * _C,), jnp.float32),
                pltpu.VMEM((_C,), jnp.int32),
                pltpu.VMEM((_F * _C,), jnp.int32),
                pltpu.VMEM((_C,), jnp.float32),
                pltpu.VMEM((_CS,), jnp.float32),
            ],
            compiler_params=pltpu.CompilerParams(use_tc_tiling_on_sc=False),
        )(_a_body)
    return _sc_cache["a"](*args)


def _sc_cnt(*args):
    if "cnt" not in _sc_cache:
        _sc_cache["cnt"] = functools.partial(
            pl.kernel,
            out_type=(
                jax.ShapeDtypeStruct((_NP,), jnp.float32),
                jax.ShapeDtypeStruct((_NP,), jnp.float32),
            ),
            mesh=plsc.VectorSubcoreMesh(**_MESH),
            scratch_types=[
                pltpu.VMEM_SHARED((_NP,), jnp.float32),
                pltpu.VMEM((_CCHUNK,), jnp.int32),
                pltpu.VMEM((_CCHUNK,), jnp.float32),
                pltpu.VMEM((_CS,), jnp.float32),
            ],
            compiler_params=pltpu.CompilerParams(use_tc_tiling_on_sc=False),
        )(_cnt_body)
    return _sc_cache["cnt"](*args)


def _post_body(aw_ref, au_ref, c0_ref, c1_ref, ws_ref, bs_ref, wm_ref,
               bm_ref, wn_ref, bn_ref, g_ref, be_ref, w1_ref, b1_ref,
               w2_ref, b2_ref, o_ref):
    aw = aw_ref[...]
    au = au_ref[...]
    cnt = jnp.maximum(c0_ref[...] + c1_ref[...], 1.0)
    dot = functools.partial(lax.dot_general,
                            dimension_numbers=(((1,), (0,)), ((), ())),
                            preferred_element_type=jnp.float32)
    h1 = jax.nn.relu(dot(aw, ws_ref[...]) + bs_ref[...])
    h2 = jax.nn.relu(dot(aw, wm_ref[...]) / cnt + bm_ref[...])
    h3 = jax.nn.relu(dot(au, wn_ref[...]) + bn_ref[...])
    hh = jnp.concatenate([h1, h2, h3], axis=-1)
    mu = jnp.mean(hh, axis=-1, keepdims=True)
    var = jnp.mean((hh - mu) ** 2, axis=-1, keepdims=True)
    hh = (hh - mu) / jnp.sqrt(var + 1e-5) * g_ref[...] + be_ref[...]
    hh = jax.nn.relu(dot(hh, w1_ref[...]) + b1_ref[...])
    o_ref[...] = dot(hh, w2_ref[...]) + b2_ref[...]


def _post_call(aw, au, c0, c1, ws, bs, wm, bm, wn, bn, g, be, w1, b1, w2, b2):
    full = lambda r, c: pl.BlockSpec((r, c), lambda i: (0, 0))
    return pl.pallas_call(
        _post_body,
        grid=(_NBLK,),
        in_specs=[
            pl.BlockSpec((_BLK, _F), lambda i: (i, 0)),
            pl.BlockSpec((_BLK, _F), lambda i: (i, 0)),
            pl.BlockSpec((_BLK, 1), lambda i: (i, 0)),
            pl.BlockSpec((_BLK, 1), lambda i: (i, 0)),
            full(20, 64), full(1, 64), full(20, 64), full(1, 64),
            full(20, 64), full(1, 64), full(1, 192), full(1, 192),
            full(192, 192), full(1, 192), full(192, 5), full(1, 5),
        ],
        out_specs=pl.BlockSpec((_BLK, 5), lambda i: (i, 0)),
        out_shape=jax.ShapeDtypeStruct((_N, 5), jnp.float32),
    )(aw, au, c0, c1, ws, bs, wm, bm, wn, bn, g, be, w1, b1, w2, b2)


def kernel(h, big_features, edge_index, edge_weight,
           g_mean, be_mean, g_std, be_std, g_edge, be_edge, g_n1, be_n1,
           W_sum, b_sum, W_meanagg, b_meanagg, W_num, b_num,
           W_fc1, b_fc1, W_fc2, b_fc2):
    g = jnp.concatenate([g_mean, g_std, g_edge]).reshape(1, 15)
    be = jnp.concatenate([be_mean, be_std, be_edge]).reshape(1, 15)
    x = _pre_call(h, big_features, g, be)
    xt = jnp.pad(x.T, ((0, 0), (0, _NP - _N)))

    src = edge_index[0]
    dst = edge_index[1]
    xg = _sc_g(xt, src)
    awt, aut = _sc_a(xg, dst, edge_weight)

    dst_dep, awt = lax.optimization_barrier((dst, awt))
    c0, c1 = _sc_cnt(dst_dep)

    aw = awt.reshape(_F, _NP)[:, :_N].T
    au = aut.reshape(_F, _NP)[:, :_N].T
    c0 = c0[:_N].reshape(_N, 1)
    c1 = c1[:_N].reshape(_N, 1)
    return _post_call(
        aw, au, c0, c1,
        W_sum, b_sum.reshape(1, 64), W_meanagg, b_meanagg.reshape(1, 64),
        W_num, b_num.reshape(1, 64), g_n1.reshape(1, 192),
        be_n1.reshape(1, 192), W_fc1, b_fc1.reshape(1, 192),
        W_fc2, b_fc2.reshape(1, 5))
